# Initial kernel scaffold; baseline (speedup 1.0000x reference)
import jax
import jax.numpy as jnp
from jax.experimental import pallas as pl


def kernel(indices_2, indices_3, indices_4, tables):
    raise NotImplementedError("write your pallas kernel here")


# speedup vs baseline: 6.2400x; 6.2400x over previous
import functools

import jax
import jax.numpy as jnp
from jax import lax
from jax.experimental import pallas as pl
from jax.experimental.pallas import tpu as pltpu
from jax.experimental.pallas import tpu_sc as plsc

NUM_CORES = 2
NUM_SUBCORES = 16
NUM_WORKERS = NUM_CORES * NUM_SUBCORES


def _sc_gather(flat_tables, idx):
    num_tables, tokens = idx.shape
    rows, dim = flat_tables.shape
    per_w = tokens // NUM_WORKERS
    chunk = 640
    n_chunks = per_w // chunk
    mesh = plsc.VectorSubcoreMesh(core_axis_name="c", subcore_axis_name="s")

    @functools.partial(
        pl.kernel,
        mesh=mesh,
        out_type=jax.ShapeDtypeStruct((tokens, num_tables, dim), jnp.float32),
        compiler_params=pltpu.CompilerParams(use_tc_tiling_on_sc=False),
        scratch_types=[
            pltpu.VMEM((chunk,), jnp.int32),
            pltpu.VMEM((chunk, dim), jnp.float32),
        ],
    )
    def k(tab_hbm, idx_hbm, out_hbm, idx_v, rows_v):
        wid = lax.axis_index("s") * NUM_CORES + lax.axis_index("c")
        base0 = wid * per_w

        @pl.loop(0, n_chunks)
        def _(c):
            base = base0 + c * chunk

            @pl.loop(0, num_tables)
            def _(t):
                pltpu.sync_copy(idx_hbm.at[t, pl.ds(base, chunk)], idx_v)
                pltpu.sync_copy(tab_hbm.at[idx_v], rows_v)
                pltpu.sync_copy(rows_v, out_hbm.at[pl.ds(base, chunk), t])

    return k(flat_tables, idx)


def kernel(indices_2, indices_3, indices_4, tables):
    batch, seq, heads = indices_2.shape
    num_tables, vocab, dim = tables.shape
    tokens = batch * seq
    idx = jnp.stack([indices_2, indices_3, indices_4], axis=0)
    idx = idx.reshape(3, tokens, heads).transpose(0, 2, 1).reshape(num_tables, tokens)
    idx = idx.astype(jnp.int32) + (jnp.arange(num_tables, dtype=jnp.int32) * vocab)[:, None]
    out = _sc_gather(tables.reshape(num_tables * vocab, dim), idx)
    return out.reshape(batch, seq, num_tables * dim)


# speedup vs baseline: 6.5268x; 1.0460x over previous
import functools

import jax
import jax.numpy as jnp
from jax import lax
from jax.experimental import pallas as pl
from jax.experimental.pallas import tpu as pltpu
from jax.experimental.pallas import tpu_sc as plsc

NUM_CORES = 2
NUM_SUBCORES = 16
NUM_WORKERS = NUM_CORES * NUM_SUBCORES
CHUNK = 640


def _sc_gather(flat_tables, idx):
    num_tables, tokens = idx.shape
    _, dim = flat_tables.shape
    per_w = tokens // NUM_WORKERS
    n_chunks = per_w // CHUNK
    n_units = n_chunks * num_tables
    mesh = plsc.VectorSubcoreMesh(core_axis_name="c", subcore_axis_name="s")

    @functools.partial(
        pl.kernel,
        mesh=mesh,
        out_type=jax.ShapeDtypeStruct((tokens, num_tables, dim), jnp.float32),
        compiler_params=pltpu.CompilerParams(use_tc_tiling_on_sc=False),
        scratch_types=[
            pltpu.VMEM((2, CHUNK), jnp.int32),
            pltpu.VMEM((2, CHUNK, dim), jnp.float32),
            pltpu.SemaphoreType.DMA,
            pltpu.SemaphoreType.DMA,
            pltpu.SemaphoreType.DMA,
            pltpu.SemaphoreType.DMA,
        ],
    )
    def k(tab_hbm, idx_hbm, out_hbm, idx_v, rows_v, g0, g1, w0, w1):
        gsem = (g0, g1)
        wsem = (w0, w1)
        wid = lax.axis_index("s") * NUM_CORES + lax.axis_index("c")
        base0 = wid * per_w

        def unit(u):
            return base0 + (u // num_tables) * CHUNK, u % num_tables

        def start_gather(u, b):
            base, t = unit(u)
            pltpu.sync_copy(idx_hbm.at[t, pl.ds(base, CHUNK)], idx_v.at[b])
            pltpu.async_copy(tab_hbm.at[idx_v.at[b]], rows_v.at[b], gsem[b])

        def wait_gather(b):
            pltpu.make_async_copy(tab_hbm.at[idx_v.at[b]], rows_v.at[b], gsem[b]).wait()

        def start_write(u, b):
            base, t = unit(u)
            pltpu.async_copy(rows_v.at[b], out_hbm.at[pl.ds(base, CHUNK), t], wsem[b])

        def wait_write(b):
            base, t = unit(0)
            pltpu.make_async_copy(rows_v.at[b], out_hbm.at[pl.ds(base, CHUNK), t], wsem[b]).wait()

        start_gather(0, 0)

        @pl.loop(0, n_units // 2)
        def _(i):
            u = 2 * i

            @pl.when(i >= 1)
            def _():
                wait_write(1)

            start_gather(u + 1, 1)
            wait_gather(0)
            start_write(u, 0)

            @pl.when(i < n_units // 2 - 1)
            def _():
                wait_write(0)
                start_gather(u + 2, 0)

            wait_gather(1)
            start_write(u + 1, 1)

        wait_write(0)
        wait_write(1)

    return k(flat_tables, idx)


def kernel(indices_2, indices_3, indices_4, tables):
    batch, seq, heads = indices_2.shape
    num_tables, vocab, dim = tables.shape
    tokens = batch * seq
    idx = jnp.stack([indices_2, indices_3, indices_4], axis=0)
    idx = idx.reshape(3, tokens, heads).transpose(0, 2, 1).reshape(num_tables, tokens)
    idx = idx.astype(jnp.int32) + (jnp.arange(num_tables, dtype=jnp.int32) * vocab)[:, None]
    out = _sc_gather(tables.reshape(num_tables * vocab, dim), idx)
    return out.reshape(batch, seq, num_tables * dim)
